# baseline (speedup 1.0000x reference)
import functools

import jax
import jax.numpy as jnp
from jax import lax
from jax.experimental import pallas as pl
from jax.experimental.pallas import tpu as pltpu
from jax.experimental.pallas import tpu_sc as plsc

VOCAB = 1000000
EMB = 64
BATCH = 4096
SEQ = 200
NCLS = 128

NC = 2
NS = 16
NW = NC * NS
BPW = BATCH // NW
CHL0 = 104
CHL1 = SEQ - CHL0
LANES = 16
KV = EMB // LANES


def _sc_pool(x_flat, table):
    mesh = plsc.VectorSubcoreMesh(core_axis_name="c", subcore_axis_name="s")

    @functools.partial(
        pl.kernel,
        out_type=jax.ShapeDtypeStruct((BATCH, EMB), jnp.float32),
        mesh=mesh,
        compiler_params=pltpu.CompilerParams(use_tc_tiling_on_sc=False),
        scratch_types=[
            pltpu.VMEM((BPW * SEQ,), jnp.int32),
            pltpu.VMEM((SEQ, EMB), jnp.float32),
            pltpu.VMEM((BPW, EMB), jnp.float32),
            pltpu.SemaphoreType.DMA,
        ],
    )
    def k(x_hbm, tab_hbm, out_hbm, idx_v, rows_v, out_v, sem):
        wid = lax.axis_index("s") * NC + lax.axis_index("c")
        base = wid * BPW
        pltpu.sync_copy(x_hbm.at[pl.ds(base * SEQ, BPW * SEQ)], idx_v)

        def body(b, carry):
            off = b * SEQ
            g0 = pltpu.async_copy(
                tab_hbm.at[idx_v.at[pl.ds(off, CHL0)]],
                rows_v.at[pl.ds(0, CHL0)], sem)
            g1 = pltpu.async_copy(
                tab_hbm.at[idx_v.at[pl.ds(off + CHL0, CHL1)]],
                rows_v.at[pl.ds(CHL0, CHL1)], sem)
            g0.wait()
            g1.wait()

            def red(j, accs):
                return tuple(a + rows_v[j, pl.ds(LANES * kk, LANES)]
                             for kk, a in enumerate(accs))

            accs = lax.fori_loop(
                0, SEQ, red,
                tuple(jnp.zeros((LANES,), jnp.float32) for _ in range(KV)),
                unroll=8)
            for kk in range(KV):
                out_v[b, pl.ds(LANES * kk, LANES)] = accs[kk]
            return carry

        lax.fori_loop(0, BPW, body, 0)
        pltpu.sync_copy(out_v, out_hbm.at[pl.ds(base, BPW)])

    return k(x_flat, table)


def _tc_head(pooled, x, row0, xlf, W2, b2):
    R = 512

    def body(p_ref, x_ref, r0_ref, xl_ref, w_ref, b_ref, o_ref):
        n0 = jnp.sum(jnp.where(x_ref[...] == 0, 1.0, 0.0),
                     axis=1, keepdims=True)
        e = (p_ref[...] - n0 * r0_ref[...]) / xl_ref[...]
        logits = jnp.dot(e, w_ref[...], preferred_element_type=jnp.float32)
        logits = logits + b_ref[...]
        m = jnp.max(logits, axis=1, keepdims=True)
        z = logits - m
        lse = jnp.log(jnp.sum(jnp.exp(z), axis=1, keepdims=True))
        o_ref[...] = z - lse

    return pl.pallas_call(
        body,
        grid=(BATCH // R,),
        in_specs=[
            pl.BlockSpec((R, EMB), lambda i: (i, 0)),
            pl.BlockSpec((R, SEQ), lambda i: (i, 0)),
            pl.BlockSpec((1, EMB), lambda i: (0, 0)),
            pl.BlockSpec((R, 1), lambda i: (i, 0)),
            pl.BlockSpec((EMB, NCLS), lambda i: (0, 0)),
            pl.BlockSpec((1, NCLS), lambda i: (0, 0)),
        ],
        out_specs=pl.BlockSpec((R, NCLS), lambda i: (i, 0)),
        out_shape=jax.ShapeDtypeStruct((BATCH, NCLS), jnp.float32),
    )(pooled, x, row0, xlf, W2, b2)


def kernel(x, x_len, table, gamma, beta, running_mean, running_var, W_fc, b_fc):
    pooled = _sc_pool(x.reshape(BATCH * SEQ), table)
    row0 = lax.slice(table, (0, 0), (1, EMB))
    s = gamma * lax.rsqrt(running_var + 1e-5)
    t = beta - running_mean * s
    W2 = (W_fc * s[None, :]).T
    b2 = (t @ W_fc.T + b_fc)[None, :]
    xlf = x_len.astype(jnp.float32).reshape(BATCH, 1)
    return _tc_head(pooled, x, row0, xlf, W2, b2)


# speedup vs baseline: 1.1324x; 1.1324x over previous
import functools

import jax
import jax.numpy as jnp
from jax import lax
from jax.experimental import pallas as pl
from jax.experimental.pallas import tpu as pltpu
from jax.experimental.pallas import tpu_sc as plsc

VOCAB = 1000000
EMB = 64
BATCH = 4096
SEQ = 200
NCLS = 128

NC = 2
NS = 16
NW = NC * NS
BPW = BATCH // NW
CHL0 = 104
CHL1 = SEQ - CHL0
LANES = 16
KV = EMB // LANES


def _sc_pool(x_flat, table):
    mesh = plsc.VectorSubcoreMesh(core_axis_name="c", subcore_axis_name="s")

    @functools.partial(
        pl.kernel,
        out_type=jax.ShapeDtypeStruct((BATCH, EMB), jnp.float32),
        mesh=mesh,
        compiler_params=pltpu.CompilerParams(use_tc_tiling_on_sc=False),
        scratch_types=[
            pltpu.VMEM((BPW * SEQ,), jnp.int32),
            pltpu.VMEM((SEQ, EMB), jnp.float32),
            pltpu.VMEM((SEQ, EMB), jnp.float32),
            pltpu.VMEM((BPW, EMB), jnp.float32),
            pltpu.SemaphoreType.DMA,
            pltpu.SemaphoreType.DMA,
        ],
    )
    def k(x_hbm, tab_hbm, out_hbm, idx_v, rows_a, rows_b, out_v, sem_a, sem_b):
        wid = lax.axis_index("s") * NC + lax.axis_index("c")
        base = wid * BPW
        pltpu.sync_copy(x_hbm.at[pl.ds(base * SEQ, BPW * SEQ)], idx_v)

        def copies(b, rows, sem):
            off = b * SEQ
            return (
                pltpu.make_async_copy(
                    tab_hbm.at[idx_v.at[pl.ds(off, CHL0)]],
                    rows.at[pl.ds(0, CHL0)], sem),
                pltpu.make_async_copy(
                    tab_hbm.at[idx_v.at[pl.ds(off + CHL0, CHL1)]],
                    rows.at[pl.ds(CHL0, CHL1)], sem),
            )

        def start(b, rows, sem):
            for c in copies(b, rows, sem):
                c.start()

        def wait(b, rows, sem):
            for c in copies(b, rows, sem):
                c.wait()

        def reduce(b, rows):
            def red(j, accs):
                out = []
                for g in range(2):
                    r = 2 * j + g
                    for kk in range(KV):
                        out.append(accs[g * KV + kk]
                                   + rows[r, pl.ds(LANES * kk, LANES)])
                return tuple(out)

            accs = lax.fori_loop(
                0, SEQ // 2, red,
                tuple(jnp.zeros((LANES,), jnp.float32) for _ in range(2 * KV)),
                unroll=4)
            for kk in range(KV):
                out_v[b, pl.ds(LANES * kk, LANES)] = accs[kk] + accs[KV + kk]

        start(0, rows_a, sem_a)

        def body(i, carry):
            b0 = 2 * i
            start(b0 + 1, rows_b, sem_b)
            wait(b0, rows_a, sem_a)
            reduce(b0, rows_a)

            @pl.when(b0 + 2 < BPW)
            def _():
                start(b0 + 2, rows_a, sem_a)

            wait(b0 + 1, rows_b, sem_b)
            reduce(b0 + 1, rows_b)
            return carry

        lax.fori_loop(0, BPW // 2, body, 0)
        pltpu.sync_copy(out_v, out_hbm.at[pl.ds(base, BPW)])

    return k(x_flat, table)


def _tc_head(pooled, x, row0, xlf, W2, b2):
    R = 512

    def body(p_ref, x_ref, r0_ref, xl_ref, w_ref, b_ref, o_ref):
        n0 = jnp.sum(jnp.where(x_ref[...] == 0, 1.0, 0.0),
                     axis=1, keepdims=True)
        e = (p_ref[...] - n0 * r0_ref[...]) / xl_ref[...]
        logits = jnp.dot(e, w_ref[...], preferred_element_type=jnp.float32)
        logits = logits + b_ref[...]
        m = jnp.max(logits, axis=1, keepdims=True)
        z = logits - m
        lse = jnp.log(jnp.sum(jnp.exp(z), axis=1, keepdims=True))
        o_ref[...] = z - lse

    return pl.pallas_call(
        body,
        grid=(BATCH // R,),
        in_specs=[
            pl.BlockSpec((R, EMB), lambda i: (i, 0)),
            pl.BlockSpec((R, SEQ), lambda i: (i, 0)),
            pl.BlockSpec((1, EMB), lambda i: (0, 0)),
            pl.BlockSpec((R, 1), lambda i: (i, 0)),
            pl.BlockSpec((EMB, NCLS), lambda i: (0, 0)),
            pl.BlockSpec((1, NCLS), lambda i: (0, 0)),
        ],
        out_specs=pl.BlockSpec((R, NCLS), lambda i: (i, 0)),
        out_shape=jax.ShapeDtypeStruct((BATCH, NCLS), jnp.float32),
    )(pooled, x, row0, xlf, W2, b2)


def kernel(x, x_len, table, gamma, beta, running_mean, running_var, W_fc, b_fc):
    pooled = _sc_pool(x.reshape(BATCH * SEQ), table)
    row0 = lax.slice(table, (0, 0), (1, EMB))
    s = gamma * lax.rsqrt(running_var + 1e-5)
    t = beta - running_mean * s
    W2 = (W_fc * s[None, :]).T
    b2 = (t @ W_fc.T + b_fc)[None, :]
    xlf = x_len.astype(jnp.float32).reshape(BATCH, 1)
    return _tc_head(pooled, x, row0, xlf, W2, b2)


# speedup vs baseline: 1.1368x; 1.0039x over previous
import functools

import jax
import jax.numpy as jnp
from jax import lax
from jax.experimental import pallas as pl
from jax.experimental.pallas import tpu as pltpu
from jax.experimental.pallas import tpu_sc as plsc

VOCAB = 1000000
EMB = 64
BATCH = 4096
SEQ = 200
NCLS = 128

NC = 2
NS = 16
NW = NC * NS
BPW = BATCH // NW
CHL0 = 104
CHL1 = SEQ - CHL0
LANES = 16
KV = EMB // LANES


def _sc_pool(x, table):
    mesh = plsc.VectorSubcoreMesh(core_axis_name="c", subcore_axis_name="s")

    @functools.partial(
        pl.kernel,
        out_type=jax.ShapeDtypeStruct((BATCH, EMB), jnp.float32),
        mesh=mesh,
        compiler_params=pltpu.CompilerParams(use_tc_tiling_on_sc=False),
        scratch_types=[
            pltpu.VMEM((BPW, SEQ), jnp.int32),
            pltpu.VMEM((SEQ, EMB), jnp.float32),
            pltpu.VMEM((SEQ, EMB), jnp.float32),
            pltpu.VMEM((BPW, EMB), jnp.float32),
            pltpu.SemaphoreType.DMA,
            pltpu.SemaphoreType.DMA,
        ],
    )
    def k(x_hbm, tab_hbm, out_hbm, idx_v, rows_a, rows_b, out_v, sem_a, sem_b):
        wid = lax.axis_index("s") * NC + lax.axis_index("c")
        base = wid * BPW
        pltpu.sync_copy(x_hbm.at[pl.ds(base, BPW)], idx_v)

        def copies(b, rows, sem):
            return (
                pltpu.make_async_copy(
                    tab_hbm.at[idx_v.at[b, pl.ds(0, CHL0)]],
                    rows.at[pl.ds(0, CHL0)], sem),
                pltpu.make_async_copy(
                    tab_hbm.at[idx_v.at[b, pl.ds(CHL0, CHL1)]],
                    rows.at[pl.ds(CHL0, CHL1)], sem),
            )

        def start(b, rows, sem):
            for c in copies(b, rows, sem):
                c.start()

        def wait(b, rows, sem):
            for c in copies(b, rows, sem):
                c.wait()

        def reduce(b, rows):
            def red(j, accs):
                out = []
                for g in range(2):
                    r = 2 * j + g
                    for kk in range(KV):
                        out.append(accs[g * KV + kk]
                                   + rows[r, pl.ds(LANES * kk, LANES)])
                return tuple(out)

            accs = lax.fori_loop(
                0, SEQ // 2, red,
                tuple(jnp.zeros((LANES,), jnp.float32) for _ in range(2 * KV)),
                unroll=4)
            for kk in range(KV):
                out_v[b, pl.ds(LANES * kk, LANES)] = accs[kk] + accs[KV + kk]

        start(0, rows_a, sem_a)

        def body(i, carry):
            b0 = 2 * i
            start(b0 + 1, rows_b, sem_b)
            wait(b0, rows_a, sem_a)
            reduce(b0, rows_a)

            @pl.when(b0 + 2 < BPW)
            def _():
                start(b0 + 2, rows_a, sem_a)

            wait(b0 + 1, rows_b, sem_b)
            reduce(b0 + 1, rows_b)
            return carry

        lax.fori_loop(0, BPW // 2, body, 0)
        pltpu.sync_copy(out_v, out_hbm.at[pl.ds(base, BPW)])

    return k(x, table)


def _tc_head(pooled, x, row0, xlf, W2, b2):
    R = 512

    def body(p_ref, x_ref, r0_ref, xl_ref, w_ref, b_ref, o_ref):
        n0 = jnp.sum(jnp.where(x_ref[...] == 0, 1.0, 0.0),
                     axis=1, keepdims=True)
        e = (p_ref[...] - n0 * r0_ref[...]) / xl_ref[...]
        logits = jnp.dot(e, w_ref[...], preferred_element_type=jnp.float32)
        logits = logits + b_ref[...]
        m = jnp.max(logits, axis=1, keepdims=True)
        z = logits - m
        lse = jnp.log(jnp.sum(jnp.exp(z), axis=1, keepdims=True))
        o_ref[...] = z - lse

    return pl.pallas_call(
        body,
        grid=(BATCH // R,),
        in_specs=[
            pl.BlockSpec((R, EMB), lambda i: (i, 0)),
            pl.BlockSpec((R, SEQ), lambda i: (i, 0)),
            pl.BlockSpec((1, EMB), lambda i: (0, 0)),
            pl.BlockSpec((R, 1), lambda i: (i, 0)),
            pl.BlockSpec((EMB, NCLS), lambda i: (0, 0)),
            pl.BlockSpec((1, NCLS), lambda i: (0, 0)),
        ],
        out_specs=pl.BlockSpec((R, NCLS), lambda i: (i, 0)),
        out_shape=jax.ShapeDtypeStruct((BATCH, NCLS), jnp.float32),
    )(pooled, x, row0, xlf, W2, b2)


def kernel(x, x_len, table, gamma, beta, running_mean, running_var, W_fc, b_fc):
    pooled = _sc_pool(x, table)
    row0 = lax.slice(table, (0, 0), (1, EMB))
    s = gamma * lax.rsqrt(running_var + 1e-5)
    t = beta - running_mean * s
    W2 = (W_fc * s[None, :]).T
    b2 = (t @ W_fc.T + b_fc)[None, :]
    xlf = x_len.astype(jnp.float32).reshape(BATCH, 1)
    return _tc_head(pooled, x, row0, xlf, W2, b2)
